# baseline (speedup 1.0000x reference)
import dataclasses
import functools

import jax
import jax.numpy as jnp
from jax import lax
from jax.experimental import pallas as pl
from jax.experimental.pallas import tpu as pltpu
from jax.experimental.pallas import tpu_sc as plsc

XD = 160
RK = 48
FD = 3 * RK
CHO = 27
NPTS = 1048576
NW = 32
PTS_PER_W = NPTS // NW
BB = 128
NBLK = PTS_PER_W // BB


def _tc_transpose(p_xy, p_xz, p_yz):
    def tp(a_ref, b_ref, c_ref, oa_ref, ob_ref, oc_ref):
        for j in range(8):
            sl = pl.ds(j * XD, XD)
            oa_ref[sl, :] = a_ref[:, j, :].T
            ob_ref[sl, :] = b_ref[:, j, :].T
            oc_ref[sl, :] = c_ref[:, j, :].T

    ispec = pl.BlockSpec((RK, 8, XD), lambda i: (0, i, 0))
    ospec = pl.BlockSpec((8 * XD, RK), lambda i: (i, 0))
    osd = jax.ShapeDtypeStruct((XD * XD, RK), jnp.float32)
    return pl.pallas_call(
        tp,
        grid=(XD // 8,),
        in_specs=[ispec, ispec, ispec],
        out_specs=[ospec, ospec, ospec],
        out_shape=[osd, osd, osd],
    )(p_xy, p_xz, p_yz)


def _tc_vec_transpose(v_x, v_y, v_z):
    def tp(a_ref, b_ref, c_ref, oa_ref, ob_ref, oc_ref):
        oa_ref[...] = a_ref[...].T
        ob_ref[...] = b_ref[...].T
        oc_ref[...] = c_ref[...].T

    osd = jax.ShapeDtypeStruct((XD, RK), jnp.float32)
    return pl.pallas_call(tp, out_shape=[osd, osd, osd])(v_x, v_y, v_z)


def _sc_feat(xyz, t_xy, t_xz, t_yz, v_x, v_y, v_z):
    mesh = plsc.VectorSubcoreMesh(core_axis_name="c", subcore_axis_name="s")
    cp = pltpu.CompilerParams()
    if "needs_layout_passes" in pltpu.CompilerParams.__dataclass_fields__:
        cp = dataclasses.replace(cp, needs_layout_passes=False)
    if "use_tc_tiling_on_sc" in pltpu.CompilerParams.__dataclass_fields__:
        cp = dataclasses.replace(cp, use_tc_tiling_on_sc=False)

    @functools.partial(
        pl.kernel,
        compiler_params=cp,
        out_type=jax.ShapeDtypeStruct((NPTS, FD), jnp.float32),
        mesh=mesh,
        scratch_types=[
            pltpu.VMEM((BB, 3), jnp.float32),
            pltpu.VMEM((BB,), jnp.float32),
            pltpu.VMEM((BB,), jnp.float32),
            pltpu.VMEM((BB,), jnp.float32),
            pltpu.VMEM((BB,), jnp.int32),
            pltpu.VMEM((BB,), jnp.int32),
            pltpu.VMEM((BB,), jnp.int32),
            pltpu.VMEM((BB,), jnp.int32),
            pltpu.VMEM((BB,), jnp.int32),
            pltpu.VMEM((BB,), jnp.int32),
            pltpu.VMEM((BB,), jnp.int32),
            pltpu.VMEM((BB,), jnp.int32),
            pltpu.VMEM((BB,), jnp.int32),
            pltpu.VMEM((BB,), jnp.int32),
            pltpu.VMEM((BB, RK), jnp.float32),
            pltpu.VMEM((BB, RK), jnp.float32),
            pltpu.VMEM((BB, RK), jnp.float32),
            pltpu.VMEM((BB, RK), jnp.float32),
            pltpu.VMEM((BB, FD), jnp.float32),
            pltpu.VMEM((XD, RK), jnp.float32),
            pltpu.VMEM((XD, RK), jnp.float32),
            pltpu.VMEM((XD, RK), jnp.float32),
            pltpu.SemaphoreType.DMA,
        ],
    )
    def kern(xyz_hbm, txy_hbm, txz_hbm, tyz_hbm,
             vx_hbm, vy_hbm, vz_hbm, feat_hbm,
             cv3, wxv, wyv, wzv,
             ix0v, ix1v, iy0v, iy1v, iz0v, iz1v,
             c00, c01, c10, c11, r00, r01, r10, r11,
             featv, vxt, vyt, vzt, sem):
        wid = lax.axis_index("s") * 2 + lax.axis_index("c")
        base = wid * PTS_PER_W
        iota = lax.iota(jnp.int32, 16)

        pltpu.sync_copy(vx_hbm, vxt)
        pltpu.sync_copy(vy_hbm, vyt)
        pltpu.sync_copy(vz_hbm, vzt)

        @pl.loop(0, NBLK)
        def _blk(blk):
            off = base + blk * BB
            pltpu.sync_copy(xyz_hbm.at[pl.ds(off, BB)], cv3)

            @pl.loop(0, BB, step=16)
            def _axes(i):
                sl = pl.ds(i, 16)
                rows = iota + i
                for ax, (i0v, i1v, wv) in enumerate(
                        ((ix0v, ix1v, wxv),
                         (iy0v, iy1v, wyv),
                         (iz0v, iz1v, wzv))):
                    p = plsc.load_gather(
                        cv3, [rows, jnp.zeros((16,), jnp.int32) + ax])
                    f = (p + 1.0) * (0.5 * (XD - 1))
                    f = jnp.minimum(jnp.maximum(f, 0.0), float(XD - 1))
                    i0 = f.astype(jnp.int32)
                    wv[sl] = f - i0.astype(jnp.float32)
                    i0v[sl] = i0
                    i1v[sl] = jnp.minimum(i0 + 1, XD - 1)

            plane_cfg = (
                (txy_hbm, ix0v, ix1v, iy0v, iy1v, wxv, wyv,
                 iz0v, iz1v, wzv, vzt, 0),
                (txz_hbm, ix0v, ix1v, iz0v, iz1v, wxv, wzv,
                 iy0v, iy1v, wyv, vyt, RK),
                (tyz_hbm, iy0v, iy1v, iz0v, iz1v, wyv, wzv,
                 ix0v, ix1v, wxv, vxt, 2 * RK),
            )
            for (pt_hbm, ih0, ih1, iw0, iw1, wh_r, ww_r,
                 jv0, jv1, wv_r, vec_t, fbase) in plane_cfg:

                @pl.loop(0, BB, step=16)
                def _cidx(i, ih0=ih0, ih1=ih1, iw0=iw0, iw1=iw1):
                    sl = pl.ds(i, 16)
                    h0 = ih0[sl] * XD
                    h1 = ih1[sl] * XD
                    w0 = iw0[sl]
                    w1 = iw1[sl]
                    c00[sl] = h0 + w0
                    c01[sl] = h0 + w1
                    c10[sl] = h1 + w0
                    c11[sl] = h1 + w1

                cp0 = pltpu.async_copy(pt_hbm.at[c00], r00, sem)
                cp1 = pltpu.async_copy(pt_hbm.at[c01], r01, sem)
                cp2 = pltpu.async_copy(pt_hbm.at[c10], r10, sem)
                cp3 = pltpu.async_copy(pt_hbm.at[c11], r11, sem)
                cp0.wait()
                cp1.wait()
                cp2.wait()
                cp3.wait()

                @pl.loop(0, BB)
                def _comb(b, wh_r=wh_r, ww_r=ww_r, jv0=jv0, jv1=jv1,
                          wv_r=wv_r, vec_t=vec_t, fbase=fbase):
                    bsel = jnp.zeros((16,), jnp.int32) + b
                    wh = plsc.load_gather(wh_r, [bsel])
                    ww = plsc.load_gather(ww_r, [bsel])
                    wv = plsc.load_gather(wv_r, [bsel])
                    j0 = plsc.load_gather(jv0, [bsel])
                    j1 = plsc.load_gather(jv1, [bsel])
                    mh = 1.0 - wh
                    mw = 1.0 - ww
                    mv = 1.0 - wv
                    w00 = mh * mw
                    w01 = mh * ww
                    w10 = wh * mw
                    w11 = wh * ww
                    for k in range(RK // 16):
                        sl = pl.ds(k * 16, 16)
                        col = iota + (k * 16)
                        acc = (r00[b, sl] * w00 + r01[b, sl] * w01
                               + r10[b, sl] * w10 + r11[b, sl] * w11)
                        u0 = plsc.load_gather(vec_t, [j0, col])
                        u1 = plsc.load_gather(vec_t, [j1, col])
                        featv[b, pl.ds(fbase + k * 16, 16)] = (
                            acc * (mv * u0 + wv * u1))

            pltpu.sync_copy(featv, feat_hbm.at[pl.ds(off, BB)])

    return kern(xyz, t_xy, t_xz, t_yz, v_x, v_y, v_z)


def _tc_matmul(feat, f_vec):
    bm = 2048

    def mm(x_ref, w_ref, o_ref):
        o_ref[...] = jnp.dot(x_ref[...], w_ref[...],
                             preferred_element_type=jnp.float32)

    return pl.pallas_call(
        mm,
        grid=(NPTS // bm,),
        in_specs=[pl.BlockSpec((bm, FD), lambda i: (i, 0)),
                  pl.BlockSpec((FD, CHO), lambda i: (0, 0))],
        out_specs=pl.BlockSpec((bm, CHO), lambda i: (i, 0)),
        out_shape=jax.ShapeDtypeStruct((NPTS, CHO), jnp.float32),
    )(feat, f_vec)


def kernel(xyz, xy_plane, xz_plane, yz_plane, x_vec, y_vec, z_vec, f_vec):
    t_xy = xy_plane[0].transpose(1, 2, 0).reshape(XD * XD, RK)
    t_xz = xz_plane[0].transpose(1, 2, 0).reshape(XD * XD, RK)
    t_yz = yz_plane[0].transpose(1, 2, 0).reshape(XD * XD, RK)
    v_x = x_vec[0, :, :, 0].T
    v_y = y_vec[0, :, :, 0].T
    v_z = z_vec[0, :, :, 0].T
    feat = _sc_feat(xyz, t_xy, t_xz, t_yz, v_x, v_y, v_z)
    return _tc_matmul(feat, f_vec)


# speedup vs baseline: 1.4720x; 1.4720x over previous
import dataclasses
import functools

import jax
import jax.numpy as jnp
from jax import lax
from jax.experimental import pallas as pl
from jax.experimental.pallas import tpu as pltpu
from jax.experimental.pallas import tpu_sc as plsc

XD = 160
RK = 48
FD = 3 * RK
CHO = 27
NPTS = 1048576
NW = 32
PTS_PER_W = NPTS // NW
BB = 128
NBLK = PTS_PER_W // BB


def _tc_transpose(p_xy, p_xz, p_yz):
    def tp(a_ref, b_ref, c_ref, oa_ref, ob_ref, oc_ref):
        for j in range(8):
            sl = pl.ds(j * XD, XD)
            oa_ref[sl, :] = a_ref[:, j, :].T
            ob_ref[sl, :] = b_ref[:, j, :].T
            oc_ref[sl, :] = c_ref[:, j, :].T

    ispec = pl.BlockSpec((RK, 8, XD), lambda i: (0, i, 0))
    ospec = pl.BlockSpec((8 * XD, RK), lambda i: (i, 0))
    osd = jax.ShapeDtypeStruct((XD * XD, RK), jnp.float32)
    return pl.pallas_call(
        tp,
        grid=(XD // 8,),
        in_specs=[ispec, ispec, ispec],
        out_specs=[ospec, ospec, ospec],
        out_shape=[osd, osd, osd],
    )(p_xy, p_xz, p_yz)


def _tc_vec_transpose(v_x, v_y, v_z):
    def tp(a_ref, b_ref, c_ref, oa_ref, ob_ref, oc_ref):
        oa_ref[...] = a_ref[...].T
        ob_ref[...] = b_ref[...].T
        oc_ref[...] = c_ref[...].T

    osd = jax.ShapeDtypeStruct((XD, RK), jnp.float32)
    return pl.pallas_call(tp, out_shape=[osd, osd, osd])(v_x, v_y, v_z)


def _sc_feat(xs, ys, zs, t_xy, t_xz, t_yz, v_x, v_y, v_z):
    mesh = plsc.VectorSubcoreMesh(core_axis_name="c", subcore_axis_name="s")
    cp = pltpu.CompilerParams()
    if "needs_layout_passes" in pltpu.CompilerParams.__dataclass_fields__:
        cp = dataclasses.replace(cp, needs_layout_passes=False)
    if "use_tc_tiling_on_sc" in pltpu.CompilerParams.__dataclass_fields__:
        cp = dataclasses.replace(cp, use_tc_tiling_on_sc=False)

    @functools.partial(
        pl.kernel,
        compiler_params=cp,
        out_type=jax.ShapeDtypeStruct((NPTS, FD), jnp.float32),
        mesh=mesh,
        scratch_types=[
            pltpu.VMEM((2, BB), jnp.float32),
            pltpu.VMEM((2, BB), jnp.float32),
            pltpu.VMEM((2, BB), jnp.float32),
            pltpu.VMEM((BB,), jnp.float32),
            pltpu.VMEM((BB,), jnp.float32),
            pltpu.VMEM((BB,), jnp.float32),
            pltpu.VMEM((BB,), jnp.int32),
            pltpu.VMEM((BB,), jnp.int32),
            pltpu.VMEM((BB,), jnp.int32),
            pltpu.VMEM((BB,), jnp.int32),
            pltpu.VMEM((BB,), jnp.int32),
            pltpu.VMEM((BB,), jnp.int32),
            pltpu.VMEM((2, 4, BB), jnp.int32),
            pltpu.VMEM((2, 4, BB, RK), jnp.float32),
            pltpu.VMEM((2, BB, FD), jnp.float32),
            pltpu.VMEM((XD, RK), jnp.float32),
            pltpu.VMEM((XD, RK), jnp.float32),
            pltpu.VMEM((XD, RK), jnp.float32),
            pltpu.SemaphoreType.DMA,
            pltpu.SemaphoreType.DMA,
            pltpu.SemaphoreType.DMA,
            pltpu.SemaphoreType.DMA,
        ],
    )
    def kern(xs_hbm, ys_hbm, zs_hbm, txy_hbm, txz_hbm, tyz_hbm,
             vx_hbm, vy_hbm, vz_hbm, feat_hbm,
             xv2, yv2, zv2, wxv, wyv, wzv,
             ix0v, ix1v, iy0v, iy1v, iz0v, iz1v,
             cidx, rows, featv, vxt, vyt, vzt,
             gsem0, gsem1, csem, fsem):
        wid = lax.axis_index("s") * 2 + lax.axis_index("c")
        base = wid * PTS_PER_W
        iota = lax.iota(jnp.int32, 16)
        gsems = (gsem0, gsem1)

        pltpu.sync_copy(vx_hbm, vxt)
        pltpu.sync_copy(vy_hbm, vyt)
        pltpu.sync_copy(vz_hbm, vzt)

        def coords_start(blk, par):
            off = base + blk * BB
            pltpu.async_copy(xs_hbm.at[pl.ds(off, BB)], xv2.at[par], csem)
            pltpu.async_copy(ys_hbm.at[pl.ds(off, BB)], yv2.at[par], csem)
            pltpu.async_copy(zs_hbm.at[pl.ds(off, BB)], zv2.at[par], csem)

        def coords_wait(par):
            pltpu.make_async_copy(xs_hbm.at[pl.ds(0, BB)], xv2.at[par],
                                  csem).wait()
            pltpu.make_async_copy(ys_hbm.at[pl.ds(0, BB)], yv2.at[par],
                                  csem).wait()
            pltpu.make_async_copy(zs_hbm.at[pl.ds(0, BB)], zv2.at[par],
                                  csem).wait()

        def plane_cfg(s):
            return ((txy_hbm, ix0v, ix1v, iy0v, iy1v, wxv, wyv,
                     iz0v, iz1v, wzv, vzt, 0),
                    (txz_hbm, ix0v, ix1v, iz0v, iz1v, wxv, wzv,
                     iy0v, iy1v, wyv, vyt, RK),
                    (tyz_hbm, iy0v, iy1v, iz0v, iz1v, wyv, wzv,
                     ix0v, ix1v, wxv, vxt, 2 * RK))[s]

        def gather_start(s, stage):
            (pt_hbm, ih0, ih1, iw0, iw1, _wh, _ww,
             _j0, _j1, _wv, _vt, _fb) = plane_cfg(s)
            cs = cidx.at[stage]

            @pl.loop(0, BB, step=16)
            def _cidx(i, ih0=ih0, ih1=ih1, iw0=iw0, iw1=iw1, cs=cs):
                sl = pl.ds(i, 16)
                h0 = ih0[sl] * XD
                h1 = ih1[sl] * XD
                w0 = iw0[sl]
                w1 = iw1[sl]
                cs[0, sl] = h0 + w0
                cs[1, sl] = h0 + w1
                cs[2, sl] = h1 + w0
                cs[3, sl] = h1 + w1

            sem = gsems[stage]
            for c in range(4):
                pltpu.async_copy(pt_hbm.at[cidx.at[stage, c]],
                                 rows.at[stage, c], sem)

        def gather_wait(s, stage):
            pt_hbm = plane_cfg(s)[0]
            sem = gsems[stage]
            for c in range(4):
                pltpu.make_async_copy(pt_hbm.at[cidx.at[stage, c]],
                                      rows.at[stage, c], sem).wait()

        def combine(s, stage, par):
            (_pt, _ih0, _ih1, _iw0, _iw1, wh_r, ww_r,
             jv0, jv1, wv_r, vec_t, fbase) = plane_cfg(s)
            r0 = rows.at[stage, 0]
            r1 = rows.at[stage, 1]
            r2 = rows.at[stage, 2]
            r3 = rows.at[stage, 3]
            fv = featv.at[par]

            @pl.loop(0, BB)
            def _comb(b, wh_r=wh_r, ww_r=ww_r, jv0=jv0, jv1=jv1,
                      wv_r=wv_r, vec_t=vec_t, fbase=fbase,
                      r0=r0, r1=r1, r2=r2, r3=r3, fv=fv):
                bsel = jnp.zeros((16,), jnp.int32) + b
                wh = plsc.load_gather(wh_r, [bsel])
                ww = plsc.load_gather(ww_r, [bsel])
                wv = plsc.load_gather(wv_r, [bsel])
                j0 = plsc.load_gather(jv0, [bsel])
                j1 = plsc.load_gather(jv1, [bsel])
                mh = 1.0 - wh
                mw = 1.0 - ww
                mv = 1.0 - wv
                w00 = mh * mw
                w01 = mh * ww
                w10 = wh * mw
                w11 = wh * ww
                for k in range(RK // 16):
                    sl = pl.ds(k * 16, 16)
                    col = iota + (k * 16)
                    acc = (r0[b, sl] * w00 + r1[b, sl] * w01
                           + r2[b, sl] * w10 + r3[b, sl] * w11)
                    u0 = plsc.load_gather(vec_t, [j0, col])
                    u1 = plsc.load_gather(vec_t, [j1, col])
                    fv[b, pl.ds(fbase + k * 16, 16)] = (
                        acc * (mv * u0 + wv * u1))

        def axes_compute(par):
            @pl.loop(0, BB, step=16)
            def _axes(i, par=par):
                sl = pl.ds(i, 16)
                for cv, i0v, i1v, wv in ((xv2, ix0v, ix1v, wxv),
                                         (yv2, iy0v, iy1v, wyv),
                                         (zv2, iz0v, iz1v, wzv)):
                    f = (cv[par, sl] + 1.0) * (0.5 * (XD - 1))
                    f = jnp.minimum(jnp.maximum(f, 0.0), float(XD - 1))
                    i0 = f.astype(jnp.int32)
                    wv[sl] = f - i0.astype(jnp.float32)
                    i0v[sl] = i0
                    i1v[sl] = jnp.minimum(i0 + 1, XD - 1)

        coords_start(0, 0)

        @pl.loop(0, NBLK, step=2)
        def _blk(blk):
            for par in range(2):
                b = blk + par
                coords_wait(par)
                nb = b + 1
                nxt = jnp.minimum(nb, NBLK - 1)
                off_n = base + nxt * BB
                pltpu.async_copy(xs_hbm.at[pl.ds(off_n, BB)],
                                 xv2.at[1 - par], csem)
                pltpu.async_copy(ys_hbm.at[pl.ds(off_n, BB)],
                                 yv2.at[1 - par], csem)
                pltpu.async_copy(zs_hbm.at[pl.ds(off_n, BB)],
                                 zv2.at[1 - par], csem)
                axes_compute(par)
                off = base + b * BB

                @pl.when(b > 1)
                def _(par=par, off=off):
                    pltpu.make_async_copy(
                        featv.at[par], feat_hbm.at[pl.ds(off, BB)],
                        fsem).wait()

                gather_start(0, 0)
                gather_start(1, 1)
                gather_wait(0, 0)
                combine(0, 0, par)
                gather_start(2, 0)
                gather_wait(1, 1)
                combine(1, 1, par)
                gather_wait(2, 0)
                combine(2, 0, par)
                pltpu.async_copy(featv.at[par], feat_hbm.at[pl.ds(off, BB)],
                                 fsem)

        coords_wait(0)
        pltpu.make_async_copy(featv.at[0], feat_hbm.at[pl.ds(0, BB)],
                              fsem).wait()
        pltpu.make_async_copy(featv.at[1], feat_hbm.at[pl.ds(0, BB)],
                              fsem).wait()

    return kern(xs, ys, zs, t_xy, t_xz, t_yz, v_x, v_y, v_z)


def _tc_matmul(feat, f_vec):
    bm = 2048

    def mm(x_ref, w_ref, o_ref):
        o_ref[...] = jnp.dot(x_ref[...], w_ref[...],
                             preferred_element_type=jnp.float32)

    return pl.pallas_call(
        mm,
        grid=(NPTS // bm,),
        in_specs=[pl.BlockSpec((bm, FD), lambda i: (i, 0)),
                  pl.BlockSpec((FD, CHO), lambda i: (0, 0))],
        out_specs=pl.BlockSpec((bm, CHO), lambda i: (i, 0)),
        out_shape=jax.ShapeDtypeStruct((NPTS, CHO), jnp.float32),
    )(feat, f_vec)


def kernel(xyz, xy_plane, xz_plane, yz_plane, x_vec, y_vec, z_vec, f_vec):
    xs = xyz[:, 0] + 0.0
    ys = xyz[:, 1] + 0.0
    zs = xyz[:, 2] + 0.0
    t_xy = xy_plane[0].transpose(1, 2, 0).reshape(XD * XD, RK)
    t_xz = xz_plane[0].transpose(1, 2, 0).reshape(XD * XD, RK)
    t_yz = yz_plane[0].transpose(1, 2, 0).reshape(XD * XD, RK)
    v_x = x_vec[0, :, :, 0].T
    v_y = y_vec[0, :, :, 0].T
    v_z = z_vec[0, :, :, 0].T
    feat = _sc_feat(xs, ys, zs, t_xy, t_xz, t_yz, v_x, v_y, v_z)
    return _tc_matmul(feat, f_vec)


# speedup vs baseline: 2.1711x; 1.4749x over previous
import dataclasses
import functools

import jax
import jax.numpy as jnp
from jax import lax
from jax.experimental import pallas as pl
from jax.experimental.pallas import tpu as pltpu
from jax.experimental.pallas import tpu_sc as plsc

XD = 160
RK = 48
FD = 3 * RK
CHO = 27
NPTS = 1048576
NW = 32
PTS_PER_W = NPTS // NW
BB = 128
NBLK = PTS_PER_W // BB


def _tc_transpose(p_xy, p_xz, p_yz):
    def tp(a_ref, b_ref, c_ref, oa_ref, ob_ref, oc_ref):
        for j in range(8):
            sl = pl.ds(j * XD, XD)
            oa_ref[sl, :] = a_ref[:, j, :].T
            ob_ref[sl, :] = b_ref[:, j, :].T
            oc_ref[sl, :] = c_ref[:, j, :].T

    ispec = pl.BlockSpec((RK, 8, XD), lambda i: (0, i, 0))
    ospec = pl.BlockSpec((8 * XD, RK), lambda i: (i, 0))
    osd = jax.ShapeDtypeStruct((XD * XD, RK), jnp.float32)
    return pl.pallas_call(
        tp,
        grid=(XD // 8,),
        in_specs=[ispec, ispec, ispec],
        out_specs=[ospec, ospec, ospec],
        out_shape=[osd, osd, osd],
    )(p_xy, p_xz, p_yz)


def _tc_vec_transpose(v_x, v_y, v_z):
    def tp(a_ref, b_ref, c_ref, oa_ref, ob_ref, oc_ref):
        oa_ref[...] = a_ref[...].T
        ob_ref[...] = b_ref[...].T
        oc_ref[...] = c_ref[...].T

    osd = jax.ShapeDtypeStruct((XD, RK), jnp.float32)
    return pl.pallas_call(tp, out_shape=[osd, osd, osd])(v_x, v_y, v_z)


def _sc_feat(xs, ys, zs, t_xy, t_xz, t_yz, v_x, v_y, v_z):
    mesh = plsc.VectorSubcoreMesh(core_axis_name="c", subcore_axis_name="s")
    cp = pltpu.CompilerParams()
    if "needs_layout_passes" in pltpu.CompilerParams.__dataclass_fields__:
        cp = dataclasses.replace(cp, needs_layout_passes=False)
    if "use_tc_tiling_on_sc" in pltpu.CompilerParams.__dataclass_fields__:
        cp = dataclasses.replace(cp, use_tc_tiling_on_sc=False)

    @functools.partial(
        pl.kernel,
        compiler_params=cp,
        out_type=jax.ShapeDtypeStruct((NPTS, FD), jnp.float32),
        mesh=mesh,
        scratch_types=[
            pltpu.VMEM((2, BB), jnp.float32),
            pltpu.VMEM((2, BB), jnp.float32),
            pltpu.VMEM((2, BB), jnp.float32),
            pltpu.VMEM((BB,), jnp.float32),
            pltpu.VMEM((BB,), jnp.float32),
            pltpu.VMEM((BB,), jnp.float32),
            pltpu.VMEM((BB,), jnp.int32),
            pltpu.VMEM((BB,), jnp.int32),
            pltpu.VMEM((BB,), jnp.int32),
            pltpu.VMEM((BB,), jnp.int32),
            pltpu.VMEM((BB,), jnp.int32),
            pltpu.VMEM((BB,), jnp.int32),
            pltpu.VMEM((2, 4, BB), jnp.int32),
            pltpu.VMEM((2, 4, BB, RK), jnp.float32),
            pltpu.VMEM((2, BB, FD), jnp.float32),
            pltpu.VMEM((XD, RK), jnp.float32),
            pltpu.VMEM((XD, RK), jnp.float32),
            pltpu.VMEM((XD, RK), jnp.float32),
            pltpu.SemaphoreType.DMA,
            pltpu.SemaphoreType.DMA,
            pltpu.SemaphoreType.DMA,
            pltpu.SemaphoreType.DMA,
        ],
    )
    def kern(xs_hbm, ys_hbm, zs_hbm, txy_hbm, txz_hbm, tyz_hbm,
             vx_hbm, vy_hbm, vz_hbm, feat_hbm,
             xv2, yv2, zv2, wxv, wyv, wzv,
             ix0v, ix1v, iy0v, iy1v, iz0v, iz1v,
             cidx, rows, featv, vxt, vyt, vzt,
             gsem0, gsem1, csem, fsem):
        wid = lax.axis_index("s") * 2 + lax.axis_index("c")
        base = wid * PTS_PER_W
        iota = lax.iota(jnp.int32, 16)
        gsems = (gsem0, gsem1)

        pltpu.sync_copy(vx_hbm, vxt)
        pltpu.sync_copy(vy_hbm, vyt)
        pltpu.sync_copy(vz_hbm, vzt)

        def coords_start(blk, par):
            off = base + blk * BB
            pltpu.async_copy(xs_hbm.at[pl.ds(off, BB)], xv2.at[par], csem)
            pltpu.async_copy(ys_hbm.at[pl.ds(off, BB)], yv2.at[par], csem)
            pltpu.async_copy(zs_hbm.at[pl.ds(off, BB)], zv2.at[par], csem)

        def coords_wait(par):
            pltpu.make_async_copy(xs_hbm.at[pl.ds(0, BB)], xv2.at[par],
                                  csem).wait()
            pltpu.make_async_copy(ys_hbm.at[pl.ds(0, BB)], yv2.at[par],
                                  csem).wait()
            pltpu.make_async_copy(zs_hbm.at[pl.ds(0, BB)], zv2.at[par],
                                  csem).wait()

        def plane_cfg(s):
            return ((txy_hbm, ix0v, ix1v, iy0v, iy1v, wxv, wyv,
                     iz0v, iz1v, wzv, vzt, 0),
                    (txz_hbm, ix0v, ix1v, iz0v, iz1v, wxv, wzv,
                     iy0v, iy1v, wyv, vyt, RK),
                    (tyz_hbm, iy0v, iy1v, iz0v, iz1v, wyv, wzv,
                     ix0v, ix1v, wxv, vxt, 2 * RK))[s]

        def gather_start(s, stage):
            (pt_hbm, ih0, ih1, iw0, iw1, _wh, _ww,
             _j0, _j1, _wv, _vt, _fb) = plane_cfg(s)
            cs = cidx.at[stage]

            @pl.loop(0, BB, step=16)
            def _cidx(i, ih0=ih0, ih1=ih1, iw0=iw0, iw1=iw1, cs=cs):
                sl = pl.ds(i, 16)
                h0 = ih0[sl] * XD
                h1 = ih1[sl] * XD
                w0 = iw0[sl]
                w1 = iw1[sl]
                cs[0, sl] = h0 + w0
                cs[1, sl] = h0 + w1
                cs[2, sl] = h1 + w0
                cs[3, sl] = h1 + w1

            sem = gsems[stage]
            for c in range(4):
                pltpu.async_copy(pt_hbm.at[cidx.at[stage, c]],
                                 rows.at[stage, c], sem)

        def gather_wait(s, stage):
            pt_hbm = plane_cfg(s)[0]
            sem = gsems[stage]
            for c in range(4):
                pltpu.make_async_copy(pt_hbm.at[cidx.at[stage, c]],
                                      rows.at[stage, c], sem).wait()

        def combine(s, stage, par):
            (_pt, _ih0, _ih1, _iw0, _iw1, wh_r, ww_r,
             jv0, jv1, wv_r, vec_t, fbase) = plane_cfg(s)
            r0 = rows.at[stage, 0]
            r1 = rows.at[stage, 1]
            r2 = rows.at[stage, 2]
            r3 = rows.at[stage, 3]
            fv = featv.at[par]

            @plsc.parallel_loop(0, BB, unroll=4)
            def _comb(b, wh_r=wh_r, ww_r=ww_r, jv0=jv0, jv1=jv1,
                      wv_r=wv_r, vec_t=vec_t, fbase=fbase,
                      r0=r0, r1=r1, r2=r2, r3=r3, fv=fv):
                bsel = jnp.zeros((16,), jnp.int32) + b
                wh = plsc.load_gather(wh_r, [bsel])
                ww = plsc.load_gather(ww_r, [bsel])
                wv = plsc.load_gather(wv_r, [bsel])
                j0 = plsc.load_gather(jv0, [bsel])
                j1 = plsc.load_gather(jv1, [bsel])
                mh = 1.0 - wh
                mw = 1.0 - ww
                mv = 1.0 - wv
                w00 = mh * mw
                w01 = mh * ww
                w10 = wh * mw
                w11 = wh * ww
                for k in range(RK // 16):
                    sl = pl.ds(k * 16, 16)
                    col = iota + (k * 16)
                    acc = (r0[b, sl] * w00 + r1[b, sl] * w01
                           + r2[b, sl] * w10 + r3[b, sl] * w11)
                    u0 = plsc.load_gather(vec_t, [j0, col])
                    u1 = plsc.load_gather(vec_t, [j1, col])
                    fv[b, pl.ds(fbase + k * 16, 16)] = (
                        acc * (mv * u0 + wv * u1))

        def axes_compute(par):
            @pl.loop(0, BB, step=16)
            def _axes(i, par=par):
                sl = pl.ds(i, 16)
                for cv, i0v, i1v, wv in ((xv2, ix0v, ix1v, wxv),
                                         (yv2, iy0v, iy1v, wyv),
                                         (zv2, iz0v, iz1v, wzv)):
                    f = (cv[par, sl] + 1.0) * (0.5 * (XD - 1))
                    f = jnp.minimum(jnp.maximum(f, 0.0), float(XD - 1))
                    i0 = f.astype(jnp.int32)
                    wv[sl] = f - i0.astype(jnp.float32)
                    i0v[sl] = i0
                    i1v[sl] = jnp.minimum(i0 + 1, XD - 1)

        coords_start(0, 0)

        @pl.loop(0, NBLK, step=2)
        def _blk(blk):
            for par in range(2):
                b = blk + par
                coords_wait(par)
                nb = b + 1
                nxt = jnp.minimum(nb, NBLK - 1)
                off_n = base + nxt * BB
                pltpu.async_copy(xs_hbm.at[pl.ds(off_n, BB)],
                                 xv2.at[1 - par], csem)
                pltpu.async_copy(ys_hbm.at[pl.ds(off_n, BB)],
                                 yv2.at[1 - par], csem)
                pltpu.async_copy(zs_hbm.at[pl.ds(off_n, BB)],
                                 zv2.at[1 - par], csem)
                axes_compute(par)
                off = base + b * BB

                @pl.when(b > 1)
                def _(par=par, off=off):
                    pltpu.make_async_copy(
                        featv.at[par], feat_hbm.at[pl.ds(off, BB)],
                        fsem).wait()

                gather_start(0, 0)
                gather_start(1, 1)
                gather_wait(0, 0)
                combine(0, 0, par)
                gather_start(2, 0)
                gather_wait(1, 1)
                combine(1, 1, par)
                gather_wait(2, 0)
                combine(2, 0, par)
                pltpu.async_copy(featv.at[par], feat_hbm.at[pl.ds(off, BB)],
                                 fsem)

        coords_wait(0)
        pltpu.make_async_copy(featv.at[0], feat_hbm.at[pl.ds(0, BB)],
                              fsem).wait()
        pltpu.make_async_copy(featv.at[1], feat_hbm.at[pl.ds(0, BB)],
                              fsem).wait()

    return kern(xs, ys, zs, t_xy, t_xz, t_yz, v_x, v_y, v_z)


def _tc_matmul(feat, f_vec):
    bm = 2048

    def mm(x_ref, w_ref, o_ref):
        o_ref[...] = jnp.dot(x_ref[...], w_ref[...],
                             preferred_element_type=jnp.float32)

    return pl.pallas_call(
        mm,
        grid=(NPTS // bm,),
        in_specs=[pl.BlockSpec((bm, FD), lambda i: (i, 0)),
                  pl.BlockSpec((FD, CHO), lambda i: (0, 0))],
        out_specs=pl.BlockSpec((bm, CHO), lambda i: (i, 0)),
        out_shape=jax.ShapeDtypeStruct((NPTS, CHO), jnp.float32),
    )(feat, f_vec)


def kernel(xyz, xy_plane, xz_plane, yz_plane, x_vec, y_vec, z_vec, f_vec):
    xs = xyz[:, 0] + 0.0
    ys = xyz[:, 1] + 0.0
    zs = xyz[:, 2] + 0.0
    t_xy = xy_plane[0].transpose(1, 2, 0).reshape(XD * XD, RK)
    t_xz = xz_plane[0].transpose(1, 2, 0).reshape(XD * XD, RK)
    t_yz = yz_plane[0].transpose(1, 2, 0).reshape(XD * XD, RK)
    v_x = x_vec[0, :, :, 0].T
    v_y = y_vec[0, :, :, 0].T
    v_z = z_vec[0, :, :, 0].T
    feat = _sc_feat(xs, ys, zs, t_xy, t_xz, t_yz, v_x, v_y, v_z)
    return _tc_matmul(feat, f_vec)


# speedup vs baseline: 2.8185x; 1.2982x over previous
import dataclasses
import functools

import jax
import jax.numpy as jnp
from jax import lax
from jax.experimental import pallas as pl
from jax.experimental.pallas import tpu as pltpu
from jax.experimental.pallas import tpu_sc as plsc

XD = 160
RK = 48
FD = 3 * RK
CHO = 27
NPTS = 1048576
NW = 32
PTS_PER_W = NPTS // NW
BB = 128
NBLK = PTS_PER_W // BB


def _tc_transpose(p_xy, p_xz, p_yz):
    def tp(a_ref, b_ref, c_ref, oa_ref, ob_ref, oc_ref):
        for j in range(8):
            sl = pl.ds(j * XD, XD)
            oa_ref[sl, :] = a_ref[:, j, :].T
            ob_ref[sl, :] = b_ref[:, j, :].T
            oc_ref[sl, :] = c_ref[:, j, :].T

    ispec = pl.BlockSpec((RK, 8, XD), lambda i: (0, i, 0))
    ospec = pl.BlockSpec((8 * XD, RK), lambda i: (i, 0))
    osd = jax.ShapeDtypeStruct((XD * XD, RK), jnp.float32)
    return pl.pallas_call(
        tp,
        grid=(XD // 8,),
        in_specs=[ispec, ispec, ispec],
        out_specs=[ospec, ospec, ospec],
        out_shape=[osd, osd, osd],
    )(p_xy, p_xz, p_yz)


def _tc_vec_transpose(v_x, v_y, v_z):
    def tp(a_ref, b_ref, c_ref, oa_ref, ob_ref, oc_ref):
        oa_ref[...] = a_ref[...].T
        ob_ref[...] = b_ref[...].T
        oc_ref[...] = c_ref[...].T

    osd = jax.ShapeDtypeStruct((XD, RK), jnp.float32)
    return pl.pallas_call(tp, out_shape=[osd, osd, osd])(v_x, v_y, v_z)


def _sc_feat(xs, ys, zs, t_xy, t_xz, t_yz, v_x, v_y, v_z):
    mesh = plsc.VectorSubcoreMesh(core_axis_name="c", subcore_axis_name="s")
    cp = pltpu.CompilerParams()
    if "needs_layout_passes" in pltpu.CompilerParams.__dataclass_fields__:
        cp = dataclasses.replace(cp, needs_layout_passes=False)
    if "use_tc_tiling_on_sc" in pltpu.CompilerParams.__dataclass_fields__:
        cp = dataclasses.replace(cp, use_tc_tiling_on_sc=False)

    @functools.partial(
        pl.kernel,
        compiler_params=cp,
        out_type=jax.ShapeDtypeStruct((NPTS // 2048 * 9, 32768), jnp.float32),
        mesh=mesh,
        scratch_types=[
            pltpu.VMEM((2, BB), jnp.float32),
            pltpu.VMEM((2, BB), jnp.float32),
            pltpu.VMEM((2, BB), jnp.float32),
            pltpu.VMEM((BB,), jnp.float32),
            pltpu.VMEM((BB,), jnp.float32),
            pltpu.VMEM((BB,), jnp.float32),
            pltpu.VMEM((BB,), jnp.int32),
            pltpu.VMEM((BB,), jnp.int32),
            pltpu.VMEM((BB,), jnp.int32),
            pltpu.VMEM((BB,), jnp.int32),
            pltpu.VMEM((BB,), jnp.int32),
            pltpu.VMEM((BB,), jnp.int32),
            pltpu.VMEM((2, 4, BB), jnp.int32),
            pltpu.VMEM((2, 4, BB, RK), jnp.float32),
            pltpu.VMEM((2, 9, 2048), jnp.float32),
            pltpu.VMEM((XD, RK), jnp.float32),
            pltpu.VMEM((XD, RK), jnp.float32),
            pltpu.VMEM((XD, RK), jnp.float32),
            pltpu.SemaphoreType.DMA,
            pltpu.SemaphoreType.DMA,
            pltpu.SemaphoreType.DMA,
            pltpu.SemaphoreType.DMA,
        ],
    )
    def kern(xs_hbm, ys_hbm, zs_hbm, txy_hbm, txz_hbm, tyz_hbm,
             vx_hbm, vy_hbm, vz_hbm, feat_hbm,
             xv2, yv2, zv2, wxv, wyv, wzv,
             ix0v, ix1v, iy0v, iy1v, iz0v, iz1v,
             cidx, rows, featv, vxt, vyt, vzt,
             gsem0, gsem1, csem, fsem):
        wid = lax.axis_index("s") * 2 + lax.axis_index("c")
        base = wid * PTS_PER_W
        iota = lax.iota(jnp.int32, 16)
        gsems = (gsem0, gsem1)

        pltpu.sync_copy(vx_hbm, vxt)
        pltpu.sync_copy(vy_hbm, vyt)
        pltpu.sync_copy(vz_hbm, vzt)

        def coords_start(blk, par):
            off = base + blk * BB
            pltpu.async_copy(xs_hbm.at[pl.ds(off, BB)], xv2.at[par], csem)
            pltpu.async_copy(ys_hbm.at[pl.ds(off, BB)], yv2.at[par], csem)
            pltpu.async_copy(zs_hbm.at[pl.ds(off, BB)], zv2.at[par], csem)

        def coords_wait(par):
            pltpu.make_async_copy(xs_hbm.at[pl.ds(0, BB)], xv2.at[par],
                                  csem).wait()
            pltpu.make_async_copy(ys_hbm.at[pl.ds(0, BB)], yv2.at[par],
                                  csem).wait()
            pltpu.make_async_copy(zs_hbm.at[pl.ds(0, BB)], zv2.at[par],
                                  csem).wait()

        def plane_cfg(s):
            return ((txy_hbm, ix0v, ix1v, iy0v, iy1v, wxv, wyv,
                     iz0v, iz1v, wzv, vzt, 0),
                    (txz_hbm, ix0v, ix1v, iz0v, iz1v, wxv, wzv,
                     iy0v, iy1v, wyv, vyt, RK),
                    (tyz_hbm, iy0v, iy1v, iz0v, iz1v, wyv, wzv,
                     ix0v, ix1v, wxv, vxt, 2 * RK))[s]

        def gather_start(s, stage):
            (pt_hbm, ih0, ih1, iw0, iw1, _wh, _ww,
             _j0, _j1, _wv, _vt, _fb) = plane_cfg(s)
            cs = cidx.at[stage]

            @pl.loop(0, BB, step=16)
            def _cidx(i, ih0=ih0, ih1=ih1, iw0=iw0, iw1=iw1, cs=cs):
                sl = pl.ds(i, 16)
                h0 = ih0[sl] * XD
                h1 = ih1[sl] * XD
                w0 = iw0[sl]
                w1 = iw1[sl]
                cs[0, sl] = h0 + w0
                cs[1, sl] = h0 + w1
                cs[2, sl] = h1 + w0
                cs[3, sl] = h1 + w1

            sem = gsems[stage]
            for c in range(4):
                pltpu.async_copy(pt_hbm.at[cidx.at[stage, c]],
                                 rows.at[stage, c], sem)

        def gather_wait(s, stage):
            pt_hbm = plane_cfg(s)[0]
            sem = gsems[stage]
            for c in range(4):
                pltpu.make_async_copy(pt_hbm.at[cidx.at[stage, c]],
                                      rows.at[stage, c], sem).wait()

        def combine(s, stage, par):
            (_pt, _ih0, _ih1, _iw0, _iw1, wh_r, ww_r,
             jv0, jv1, wv_r, vec_t, fbase) = plane_cfg(s)
            r0 = rows.at[stage, 0]
            r1 = rows.at[stage, 1]
            r2 = rows.at[stage, 2]
            r3 = rows.at[stage, 3]
            fv = featv.at[par]

            @plsc.parallel_loop(0, BB, unroll=4)
            def _comb(b, wh_r=wh_r, ww_r=ww_r, jv0=jv0, jv1=jv1,
                      wv_r=wv_r, vec_t=vec_t, fbase=fbase,
                      r0=r0, r1=r1, r2=r2, r3=r3, fv=fv):
                bsel = jnp.zeros((16,), jnp.int32) + b
                wh = plsc.load_gather(wh_r, [bsel])
                ww = plsc.load_gather(ww_r, [bsel])
                wv = plsc.load_gather(wv_r, [bsel])
                j0 = plsc.load_gather(jv0, [bsel])
                j1 = plsc.load_gather(jv1, [bsel])
                mh = 1.0 - wh
                mw = 1.0 - ww
                mv = 1.0 - wv
                w00 = mh * mw
                w01 = mh * ww
                w10 = wh * mw
                w11 = wh * ww
                u = b % 8
                lgc = (b - u) * 16
                for k in range(RK // 16):
                    sl = pl.ds(k * 16, 16)
                    col = iota + (k * 16)
                    acc = (r0[b, sl] * w00 + r1[b, sl] * w01
                           + r2[b, sl] * w10 + r3[b, sl] * w11)
                    u0 = plsc.load_gather(vec_t, [j0, col])
                    u1 = plsc.load_gather(vec_t, [j1, col])
                    pos = u * 144 + (fbase + k * 16)
                    tt = pos // 128
                    cc = (pos % 128) + lgc
                    fv[tt, pl.ds(cc, 16)] = (
                        acc * (mv * u0 + wv * u1))

        def axes_compute(par):
            @pl.loop(0, BB, step=16)
            def _axes(i, par=par):
                sl = pl.ds(i, 16)
                for cv, i0v, i1v, wv in ((xv2, ix0v, ix1v, wxv),
                                         (yv2, iy0v, iy1v, wyv),
                                         (zv2, iz0v, iz1v, wzv)):
                    f = (cv[par, sl] + 1.0) * (0.5 * (XD - 1))
                    f = jnp.minimum(jnp.maximum(f, 0.0), float(XD - 1))
                    i0 = f.astype(jnp.int32)
                    wv[sl] = f - i0.astype(jnp.float32)
                    i0v[sl] = i0
                    i1v[sl] = jnp.minimum(i0 + 1, XD - 1)

        coords_start(0, 0)

        @pl.loop(0, NBLK, step=2)
        def _blk(blk):
            for par in range(2):
                b = blk + par
                coords_wait(par)
                nb = b + 1
                nxt = jnp.minimum(nb, NBLK - 1)
                off_n = base + nxt * BB
                pltpu.async_copy(xs_hbm.at[pl.ds(off_n, BB)],
                                 xv2.at[1 - par], csem)
                pltpu.async_copy(ys_hbm.at[pl.ds(off_n, BB)],
                                 yv2.at[1 - par], csem)
                pltpu.async_copy(zs_hbm.at[pl.ds(off_n, BB)],
                                 zv2.at[1 - par], csem)
                axes_compute(par)
                off = base + b * BB

                @pl.when(b > 1)
                def _(par=par):
                    pltpu.make_async_copy(
                        featv.at[par],
                        feat_hbm.at[pl.ds(0, 9), pl.ds(0, 2048)],
                        fsem).wait()

                gather_start(0, 0)
                gather_start(1, 1)
                gather_wait(0, 0)
                combine(0, 0, par)
                gather_start(2, 0)
                gather_wait(1, 1)
                combine(1, 1, par)
                gather_wait(2, 0)
                combine(2, 0, par)
                sb = off // 2048
                i0c = (off % 2048) * 16
                pltpu.async_copy(
                    featv.at[par],
                    feat_hbm.at[pl.ds(sb * 9, 9), pl.ds(i0c, 2048)],
                    fsem)

        coords_wait(0)
        pltpu.make_async_copy(featv.at[0],
                              feat_hbm.at[pl.ds(0, 9), pl.ds(0, 2048)],
                              fsem).wait()
        pltpu.make_async_copy(featv.at[1],
                              feat_hbm.at[pl.ds(0, 9), pl.ds(0, 2048)],
                              fsem).wait()

    return kern(xs, ys, zs, t_xy, t_xz, t_yz, v_x, v_y, v_z)


def _tc_matmul(feat, f_vec):
    fv4 = feat.reshape(512, 9, 256, 128)
    ws = []
    for t in range(9):
        s = t * 128
        a = s // 144
        j0 = s - a * 144
        n1 = min(128, 144 - j0)
        w1 = jnp.zeros((128, CHO), f_vec.dtype).at[:n1].set(
            f_vec[j0:j0 + n1])
        w2 = jnp.zeros((128, CHO), f_vec.dtype)
        if n1 < 128:
            w2 = w2.at[n1:].set(f_vec[:128 - n1])
        ws += [w1, w2]
    wcat = jnp.concatenate(ws, axis=0)

    def mm(x_ref, w_ref, o_ref):
        outs = [None] * 8
        for t in range(9):
            xt = x_ref[0, t]
            s = t * 128
            a = s // 144
            n1 = min(128, 144 - (s - a * 144))
            w1 = w_ref[pl.ds((2 * t) * 128, 128), :]
            y1 = jnp.dot(xt, w1, preferred_element_type=jnp.float32)
            outs[a] = y1 if outs[a] is None else outs[a] + y1
            if n1 < 128:
                w2 = w_ref[pl.ds((2 * t + 1) * 128, 128), :]
                y2 = jnp.dot(xt, w2, preferred_element_type=jnp.float32)
                outs[a + 1] = (y2 if outs[a + 1] is None
                               else outs[a + 1] + y2)
        for a in range(8):
            o_ref[:, a, :] = outs[a]

    out = pl.pallas_call(
        mm,
        grid=(512,),
        in_specs=[pl.BlockSpec((1, 9, 256, 128), lambda i: (i, 0, 0, 0)),
                  pl.BlockSpec((2304, CHO), lambda i: (0, 0))],
        out_specs=pl.BlockSpec((256, 8, CHO), lambda i: (i, 0, 0)),
        out_shape=jax.ShapeDtypeStruct((NPTS // 8, 8, CHO), jnp.float32),
    )(fv4, wcat)
    return out.reshape(NPTS, CHO)


def kernel(xyz, xy_plane, xz_plane, yz_plane, x_vec, y_vec, z_vec, f_vec):
    xs = xyz[:, 0] + 0.0
    ys = xyz[:, 1] + 0.0
    zs = xyz[:, 2] + 0.0
    t_xy = xy_plane[0].transpose(1, 2, 0).reshape(XD * XD, RK)
    t_xz = xz_plane[0].transpose(1, 2, 0).reshape(XD * XD, RK)
    t_yz = yz_plane[0].transpose(1, 2, 0).reshape(XD * XD, RK)
    v_x = x_vec[0, :, :, 0].T
    v_y = y_vec[0, :, :, 0].T
    v_z = z_vec[0, :, :, 0].T
    feat = _sc_feat(xs, ys, zs, t_xy, t_xz, t_yz, v_x, v_y, v_z)
    return _tc_matmul(feat, f_vec)
